# baseline (speedup 1.0000x reference)
import jax
import jax.numpy as jnp
from jax import lax
from jax.experimental import pallas as pl
from jax.experimental.pallas import tpu as pltpu
from jax.experimental.pallas import tpu_sc as plsc

_VOCAB = 400000
_EMBED = 200
_BATCH = 1024
_SEQ = 200
_N = _BATCH * _SEQ
_E8 = _EMBED // 8
_BT = _BATCH // 128

_INFO = plsc.get_sparse_core_info()
_NC = _INFO.num_cores
_NS = _INFO.num_subcores
_NW = _NC * _NS
_PER_W = _N // _NW
_CHUNK = 128
_NCH = _PER_W // _CHUNK


def _glove_body(batch_hbm, idmap_hbm, tablea_hbm, tableb_hbm, out_hbm,
                bidx, gidx, rowsa, rowsb, cols, sem_id,
                g0, g1, h0, h1, c0, c1):
    gsems = (g0, g1)
    hsems = (h0, h1)
    csems = (c0, c1)
    wid = lax.axis_index("s") * _NC + lax.axis_index("c")
    base = wid * _PER_W

    pltpu.sync_copy(batch_hbm.at[pl.ds(base, _PER_W)], bidx)

    def fire_ids(j, c):
        sl = pl.ds(j * _CHUNK, _CHUNK)
        pltpu.async_copy(idmap_hbm.at[bidx.at[sl]], gidx.at[sl], sem_id)
        return c

    lax.fori_loop(0, _NCH, fire_ids, 0)
    pltpu.make_async_copy(batch_hbm.at[pl.ds(base, _PER_W)], gidx, sem_id).wait()

    lanes = lax.iota(jnp.int32, 16)

    def transpose_slab(b):
        bv = jnp.full((16,), 0, jnp.int32)

        def per_i(i, c):
            iv = lanes * 0 + i
            for ec in range(8):
                ev = lanes + 16 * ec
                v = rowsa[b, i, pl.ds(16 * ec, 16)]
                plsc.store_scatter(cols,
                                   [bv, ev >> 3, ev & 7, iv], v, mask=None)
            for ec in range(5):
                off = 16 * ec if ec < 4 else 72 - 16
                ev = lanes + 128 + off
                v = rowsb[b, i, pl.ds(off, 16)]
                mask = None if ec < 4 else (lanes >= 8)
                plsc.store_scatter(cols,
                                   [bv, ev >> 3, ev & 7, iv], v, mask=mask)
            return c
        lax.fori_loop(0, _CHUNK, per_i, 0)

    for b in range(2):
        sl = pl.ds(b * _CHUNK, _CHUNK)
        pltpu.async_copy(tablea_hbm.at[gidx.at[sl]], rowsa.at[b], gsems[b])
        pltpu.async_copy(tableb_hbm.at[gidx.at[sl]], rowsb.at[b], hsems[b])

    def step(g, c):
        for b in range(2):
            j = 2 * g + b
            q = base // _CHUNK + j
            s = q // _BT
            bt = q - s * _BT
            sl = pl.ds(j * _CHUNK, _CHUNK)
            pltpu.make_async_copy(tablea_hbm.at[gidx.at[sl]],
                                  rowsa.at[b], gsems[b]).wait()
            pltpu.make_async_copy(tableb_hbm.at[gidx.at[sl]],
                                  rowsb.at[b], hsems[b]).wait()

            @pl.when(j >= 1)
            def _():
                pltpu.make_async_copy(cols.at[0, :, :, pl.ds(0, 128)],
                                      out_hbm.at[s, :, bt], csems[0]).wait()

            transpose_slab(b)

            @pl.when(j + 2 < _NCH)
            def _():
                sl2 = pl.ds((j + 2) * _CHUNK, _CHUNK)
                pltpu.async_copy(tablea_hbm.at[gidx.at[sl2]],
                                 rowsa.at[b], gsems[b])
                pltpu.async_copy(tableb_hbm.at[gidx.at[sl2]],
                                 rowsb.at[b], hsems[b])

            pltpu.async_copy(cols.at[0, :, :, pl.ds(0, 128)],
                             out_hbm.at[s, :, bt], csems[0])
        return c

    lax.fori_loop(0, _NCH // 2, step, 0)

    pltpu.make_async_copy(cols.at[0, :, :, pl.ds(0, 128)],
                          out_hbm.at[0, :, 0], csems[0]).wait()


_glove_call = pl.kernel(
    _glove_body,
    out_type=jax.ShapeDtypeStruct((_SEQ, _E8, _BT, 8, 128), jnp.float32),
    mesh=plsc.VectorSubcoreMesh(core_axis_name="c", subcore_axis_name="s"),
    scratch_types=[
        pltpu.VMEM((_PER_W,), jnp.int32),
        pltpu.VMEM((_PER_W,), jnp.int32),
        pltpu.VMEM((2, _CHUNK, 128), jnp.float32),
        pltpu.VMEM((2, _CHUNK, 72), jnp.float32),
        pltpu.VMEM((1, _E8, 8, 129), jnp.float32),
    ] + [pltpu.SemaphoreType.DMA] * 7,
    compiler_params=pltpu.CompilerParams(use_tc_tiling_on_sc=False,
                                         needs_layout_passes=False),
)


@jax.jit
def kernel(batch, id_map, table):
    flat = batch.T.reshape(_N).astype(jnp.int32)
    id_map = id_map.astype(jnp.int32)
    ta, tb = jax.lax.optimization_barrier((table[:, :128], table[:, 128:200]))
    out5d = _glove_call(flat, id_map, ta, tb)
    return out5d.transpose(2, 4, 0, 1, 3).reshape(_BATCH, _SEQ, _EMBED)


# speedup vs baseline: 1.0759x; 1.0759x over previous
import jax
import jax.numpy as jnp
from jax import lax
from jax.experimental import pallas as pl
from jax.experimental.pallas import tpu as pltpu
from jax.experimental.pallas import tpu_sc as plsc

_VOCAB = 400000
_EMBED = 200
_BATCH = 1024
_SEQ = 200
_N = _BATCH * _SEQ
_E8 = _EMBED // 8
_BT = _BATCH // 128

_INFO = plsc.get_sparse_core_info()
_NC = _INFO.num_cores
_NS = _INFO.num_subcores
_NW = _NC * _NS
_PER_W = _N // _NW
_CHUNK = 128
_NCH = _PER_W // _CHUNK


def _glove_body(batch_hbm, idmap_hbm, tablea_hbm, tableb_hbm, out_hbm,
                bidx, gidx, rowsa, rowsb, cols, sem_id,
                g0, g1, h0, h1, c0, c1):
    gsems = (g0, g1)
    hsems = (h0, h1)
    csems = (c0, c1)
    wid = lax.axis_index("s") * _NC + lax.axis_index("c")
    base = wid * _PER_W

    pltpu.sync_copy(batch_hbm.at[pl.ds(base, _PER_W)], bidx)

    def fire_ids(j, c):
        sl = pl.ds(j * _CHUNK, _CHUNK)
        pltpu.async_copy(idmap_hbm.at[bidx.at[sl]], gidx.at[sl], sem_id)
        return c

    lax.fori_loop(0, _NCH, fire_ids, 0)
    pltpu.make_async_copy(batch_hbm.at[pl.ds(base, _PER_W)], gidx, sem_id).wait()

    lanes = lax.iota(jnp.int32, 16)

    def transpose_slab(b):
        bv = jnp.full((16,), 0, jnp.int32)

        def per_i(i, c):
            iv = lanes * 0 + i
            for ec in range(8):
                ev = lanes + 16 * ec
                v = rowsa[b, i, pl.ds(16 * ec, 16)]
                plsc.store_scatter(cols,
                                   [bv, ev >> 3, ev & 7, iv], v, mask=None)
            for ec in range(5):
                off = 16 * ec if ec < 4 else 72 - 16
                ev = lanes + 128 + off
                v = rowsb[b, i, pl.ds(off, 16)]
                mask = None if ec < 4 else (lanes >= 8)
                plsc.store_scatter(cols,
                                   [bv, ev >> 3, ev & 7, iv], v, mask=mask)
            return c
        lax.fori_loop(0, _CHUNK, per_i, 0)

    for b in range(2):
        sl = pl.ds(b * _CHUNK, _CHUNK)
        pltpu.async_copy(tablea_hbm.at[gidx.at[sl]], rowsa.at[b], gsems[b])
        pltpu.async_copy(tableb_hbm.at[gidx.at[sl]], rowsb.at[b], hsems[b])

    def step(g, c):
        for b in range(2):
            j = 2 * g + b
            q = base // _CHUNK + j
            s = q // _BT
            bt = q - s * _BT
            sl = pl.ds(j * _CHUNK, _CHUNK)
            pltpu.make_async_copy(tablea_hbm.at[gidx.at[sl]],
                                  rowsa.at[b], gsems[b]).wait()
            pltpu.make_async_copy(tableb_hbm.at[gidx.at[sl]],
                                  rowsb.at[b], hsems[b]).wait()

            @pl.when(j >= 1)
            def _():
                pltpu.make_async_copy(cols.at[0, :, :, pl.ds(0, 128)],
                                      out_hbm.at[s, :, bt], csems[0]).wait()

            transpose_slab(b)

            @pl.when(j + 2 < _NCH)
            def _():
                sl2 = pl.ds((j + 2) * _CHUNK, _CHUNK)
                pltpu.async_copy(tablea_hbm.at[gidx.at[sl2]],
                                 rowsa.at[b], gsems[b])
                pltpu.async_copy(tableb_hbm.at[gidx.at[sl2]],
                                 rowsb.at[b], hsems[b])

            pltpu.async_copy(cols.at[0, :, :, pl.ds(0, 128)],
                             out_hbm.at[s, :, bt], csems[0])
        return c

    lax.fori_loop(0, _NCH // 2, step, 0)

    pltpu.make_async_copy(cols.at[0, :, :, pl.ds(0, 128)],
                          out_hbm.at[0, :, 0], csems[0]).wait()


_glove_call = pl.kernel(
    _glove_body,
    out_type=jax.ShapeDtypeStruct((_SEQ, _E8, _BT, 8, 128), jnp.float32),
    mesh=plsc.VectorSubcoreMesh(core_axis_name="c", subcore_axis_name="s"),
    scratch_types=[
        pltpu.VMEM((_PER_W,), jnp.int32),
        pltpu.VMEM((_PER_W,), jnp.int32),
        pltpu.VMEM((2, _CHUNK, 128), jnp.float32),
        pltpu.VMEM((2, _CHUNK, 72), jnp.float32),
        pltpu.VMEM((1, _E8, 8, 129), jnp.float32),
    ] + [pltpu.SemaphoreType.DMA] * 7,
    compiler_params=pltpu.CompilerParams(use_tc_tiling_on_sc=False,
                                         needs_layout_passes=False),
)


@jax.jit
def kernel(batch, id_map, table):
    flat = batch.T.reshape(_N).astype(jnp.int32)
    id_map = id_map.astype(jnp.int32)
    out5d = _glove_call(flat, id_map, table[:, :128], table[:, 128:200])
    return out5d.transpose(2, 4, 0, 1, 3).reshape(_BATCH, _SEQ, _EMBED)
